# Initial kernel scaffold; baseline (speedup 1.0000x reference)
import jax
import jax.numpy as jnp
from jax.experimental import pallas as pl


def kernel(node_emb, feature_emb, relation_index, W_gat, att_src, att_dst, bias_gat, W1, b1, g1, be1, W2, b2, g2, be2, W3, b3):
    raise NotImplementedError("write your pallas kernel here")


# speedup vs baseline: 14.9627x; 14.9627x over previous
import functools
import jax
import jax.numpy as jnp
from jax import lax
from jax.experimental import pallas as pl
from jax.experimental.pallas import tpu as pltpu
from jax.experimental.pallas import tpu_sc as plsc

BS = 2048
NF = 256
EMB = 64
H = 2
E = 8192
NW = 32
EPW = E // NW
SB = 64
NBLK = BS // SB
NPAIR = BS * NF

f32 = jnp.float32


def _prep_body(fe, wg, asw, adw, ne, w1a, x_o, as_o, ad_o, a_o):
    x = jnp.dot(fe[...], wg[...], preferred_element_type=f32)
    x_o[...] = x
    as_o[...] = jnp.dot(x, asw[...], preferred_element_type=f32)
    ad_o[...] = jnp.dot(x, adw[...], preferred_element_type=f32)
    a_o[...] = jnp.dot(ne[...], w1a[...], preferred_element_type=f32)


def _prep(fe, wg, asw, adw, ne, w1a):
    return pl.pallas_call(
        _prep_body,
        out_shape=(
            jax.ShapeDtypeStruct((NF, EMB), f32),
            jax.ShapeDtypeStruct((NF, H), f32),
            jax.ShapeDtypeStruct((NF, H), f32),
            jax.ShapeDtypeStruct((BS, EMB), f32),
        ),
    )(fe, wg, asw, adw, ne, w1a)


def _gat_sc_body(x_hbm, as_hbm, ad_hbm, se_hbm, de_hbm, num_o, den_o,
                 x_v, as_v, ad_v, se_v, de_v, num_v, den_v):
    wid = lax.axis_index("s") * 2 + lax.axis_index("c")
    base = wid * EPW

    pltpu.sync_copy(x_hbm, x_v)
    pltpu.sync_copy(as_hbm, as_v)
    pltpu.sync_copy(ad_hbm, ad_v)
    pltpu.sync_copy(se_hbm.at[pl.ds(base, EPW)], se_v)
    pltpu.sync_copy(de_hbm.at[pl.ds(base, EPW)], de_v)

    z16 = jnp.zeros((16,), f32)

    def _zero_num(i, _):
        num_v[pl.ds(pl.multiple_of(i * 16, 16), 16)] = z16
        return 0
    lax.fori_loop(0, (NF * EMB) // 16, _zero_num, 0)

    def _zero_den(i, _):
        den_v[pl.ds(pl.multiple_of(i * 16, 16), 16)] = z16
        return 0
    lax.fori_loop(0, (NF * H) // 16, _zero_den, 0)

    def _edges(g, _):
        off = pl.multiple_of(g * 16, 16)
        sv = se_v[pl.ds(off, 16)]
        dv = de_v[pl.ds(off, 16)]
        for h in range(H):
            asg = plsc.load_gather(as_v, [sv * H + h])
            adg = plsc.load_gather(ad_v, [dv * H + h])
            al = asg + adg
            al = jnp.where(al > 0, al, al * 0.2)
            ex = jnp.exp(al)
            plsc.addupdate_scatter(den_v, [dv * H + h], ex)
            for j in range(32):
                col = h * 32 + j
                xg = plsc.load_gather(x_v, [sv * EMB + col])
                plsc.addupdate_scatter(num_v, [dv * EMB + col], xg * ex)
        return 0
    lax.fori_loop(0, EPW // 16, _edges, 0)

    pltpu.sync_copy(num_v, num_o.at[wid])
    pltpu.sync_copy(den_v, den_o.at[wid])


def _gat_sc(x, a_src, a_dst, se, de):
    mesh = plsc.VectorSubcoreMesh(core_axis_name="c", subcore_axis_name="s")
    fn = pl.kernel(
        _gat_sc_body, mesh=mesh,
        out_type=(
            jax.ShapeDtypeStruct((NW, NF * EMB), f32),
            jax.ShapeDtypeStruct((NW, NF * H), f32),
        ),
        scratch_types=[
            pltpu.VMEM((NF * EMB,), f32),
            pltpu.VMEM((NF * H,), f32),
            pltpu.VMEM((NF * H,), f32),
            pltpu.VMEM((EPW,), jnp.int32),
            pltpu.VMEM((EPW,), jnp.int32),
            pltpu.VMEM((NF * EMB,), f32),
            pltpu.VMEM((NF * H,), f32),
        ],
        compiler_params=pltpu.CompilerParams(needs_layout_passes=False),
    )
    return fn(x.reshape(NF * EMB), a_src.reshape(NF * H), a_dst.reshape(NF * H),
              se, de)


def _lrelu(z, s):
    return jnp.where(z > 0, z, z * s)


def _fold_prep(a_r, np_r, dp_r, bias_r, w1b_r, at_s, bt_s):
    nsum = np_r[0]
    dsum = dp_r[0]
    for w in range(1, NW):
        nsum = nsum + np_r[w]
        dsum = dsum + dp_r[w]
    denb = jnp.concatenate(
        [jnp.broadcast_to(dsum[:, 0:1], (NF, 32)),
         jnp.broadcast_to(dsum[:, 1:2], (NF, 32))], axis=1)
    msg = nsum / (denb + 1e-16) + bias_r[...]
    b = jnp.dot(msg, w1b_r[...], preferred_element_type=f32)
    a = a_r[...]
    am = jnp.mean(a, axis=0, keepdims=True)
    av = jnp.mean(a * a, axis=0, keepdims=True) - am * am
    bm = jnp.mean(b, axis=0, keepdims=True)
    bv = jnp.mean(b * b, axis=0, keepdims=True) - bm * bm
    s1 = jax.lax.rsqrt(av + bv + 1e-5)
    at_s[...] = (a - am) * s1
    bt_s[...] = (b - bm) * s1


def _m1_body(a_r, np_r, dp_r, bias_r, w1b_r, g1_r, be1_r, w2_r,
             stats_o, at_o, bt_o, at_s, bt_s):
    i = pl.program_id(0)

    @pl.when(i == 0)
    def _():
        _fold_prep(a_r, np_r, dp_r, bias_r, w1b_r, at_s, bt_s)
        stats_o[...] = jnp.zeros((2, 32), f32)

    ablk = at_s[pl.ds(i * SB, SB), :] * g1_r[...]
    bt = bt_s[...] * g1_r[...]
    z = (ablk[:, None, :] + bt[None, :, :]) + be1_r[...]
    h1 = _lrelu(z, 0.01).reshape(SB * NF, EMB)
    y = jnp.dot(h1, w2_r[...], preferred_element_type=f32)
    sy = jnp.sum(y, axis=0, keepdims=True)
    syy = jnp.sum(y * y, axis=0, keepdims=True)
    stats_o[...] = stats_o[...] + jnp.concatenate([sy, syy], axis=0)
    at_o[...] = at_s[pl.ds(i * SB, SB), :]
    bt_o[...] = bt_s[...]


def _m1(a, nparts, dparts, bias_gat, w1b, g1, be1, w2):
    full = lambda s: pl.BlockSpec(s, lambda i: tuple(0 for _ in s))
    return pl.pallas_call(
        _m1_body,
        grid=(NBLK,),
        in_specs=[
            full((BS, EMB)),
            full((NW, NF, EMB)),
            full((NW, NF, H)),
            full((1, EMB)),
            full((EMB, EMB)),
            full((1, EMB)),
            full((1, 1, EMB)),
            full((EMB, 32)),
        ],
        out_specs=[
            full((2, 32)),
            pl.BlockSpec((SB, EMB), lambda i: (i, 0)),
            full((NF, EMB)),
        ],
        out_shape=(
            jax.ShapeDtypeStruct((2, 32), f32),
            jax.ShapeDtypeStruct((BS, EMB), f32),
            jax.ShapeDtypeStruct((NF, EMB), f32),
        ),
        scratch_shapes=[
            pltpu.VMEM((BS, EMB), f32),
            pltpu.VMEM((NF, EMB), f32),
        ],
        compiler_params=pltpu.CompilerParams(
            dimension_semantics=("arbitrary",)),
    )(a, nparts, dparts, bias_gat, w1b, g1, be1, w2)


def _m2_body(at_r, bt_r, g1_r, be1_r, w2s_r, c2_r, w3_r, b3_r, out_o):
    ablk = at_r[...] * g1_r[...]
    bt = bt_r[...] * g1_r[...]
    z = (ablk[:, None, :] + bt[None, :, :]) + be1_r[...]
    h1 = _lrelu(z, 0.01).reshape(SB * NF, EMB)
    yp = jnp.dot(h1, w2s_r[...], preferred_element_type=f32) + c2_r[...]
    h2 = _lrelu(yp, 0.01).reshape(SB, NF, 32)
    out_o[...] = jnp.sum(h2 * w3_r[...][None, :, :], axis=-1) + b3_r[0, 0]


def _m2(at, bt, g1, be1, w2s, c2, w3, b3):
    full = lambda s: pl.BlockSpec(s, lambda i: tuple(0 for _ in s))
    return pl.pallas_call(
        _m2_body,
        grid=(NBLK,),
        in_specs=[
            pl.BlockSpec((SB, EMB), lambda i: (i, 0)),
            full((NF, EMB)),
            full((1, EMB)),
            full((1, 1, EMB)),
            full((EMB, 32)),
            full((1, 32)),
            full((1, 32)),
            full((1, 1)),
        ],
        out_specs=pl.BlockSpec((SB, NF), lambda i: (i, 0)),
        out_shape=jax.ShapeDtypeStruct((BS, NF), f32),
        compiler_params=pltpu.CompilerParams(
            dimension_semantics=("arbitrary",)),
    )(at, bt, g1, be1, w2s, c2, w3, b3)


def kernel(node_emb, feature_emb, relation_index, W_gat, att_src, att_dst,
           bias_gat, W1, b1, g1, be1, W2, b2, g2, be2, W3, b3):
    se = relation_index[0].astype(jnp.int32)
    de = relation_index[1].astype(jnp.int32)
    asw = jnp.zeros((EMB, H), f32).at[:32, 0].set(att_src[0]).at[32:, 1].set(att_src[1])
    adw = jnp.zeros((EMB, H), f32).at[:32, 0].set(att_dst[0]).at[32:, 1].set(att_dst[1])
    w1a = W1[:EMB]
    w1b = W1[EMB:]

    x, a_src, a_dst, a = _prep(feature_emb, W_gat, asw, adw, node_emb, w1a)
    nparts, dparts = _gat_sc(x, a_src, a_dst, se, de)

    stats, at, bt = _m1(
        a, nparts.reshape(NW, NF, EMB), dparts.reshape(NW, NF, H),
        bias_gat.reshape(1, EMB), w1b, g1.reshape(1, EMB),
        be1.reshape(1, 1, EMB), W2)

    my = stats[0] / NPAIR
    vy = stats[1] / NPAIR - my * my
    s2 = g2 * jax.lax.rsqrt(vy + 1e-5)
    w2s = W2 * s2[None, :]
    c2 = (be2 - my * s2).reshape(1, 32)

    out = _m2(at, bt, g1.reshape(1, EMB), be1.reshape(1, 1, EMB),
              w2s, c2, W3.reshape(1, 32), b3.reshape(1, 1))
    return out


# speedup vs baseline: 32.4046x; 2.1657x over previous
import jax
import jax.numpy as jnp
from jax import lax
from jax.experimental import pallas as pl
from jax.experimental.pallas import tpu as pltpu
from jax.experimental.pallas import tpu_sc as plsc

BS = 2048
NF = 256
EMB = 64
H = 2
E = 8192
NW = 32
EPW = E // (NW // H)
SB = 64
NBLK = BS // SB
NPAIR = BS * NF

f32 = jnp.float32


def _prep_body(fe, wg, asw, adw, ne, w1a, x_o, as_o, ad_o, a_o):
    x = jnp.dot(fe[...], wg[...], preferred_element_type=f32)
    x_o[...] = x
    as_o[...] = jnp.dot(x, asw[...], preferred_element_type=f32)
    ad_o[...] = jnp.dot(x, adw[...], preferred_element_type=f32)
    a_o[...] = jnp.dot(ne[...], w1a[...], preferred_element_type=f32)


def _prep(fe, wg, asw, adw, ne, w1a):
    return pl.pallas_call(
        _prep_body,
        out_shape=(
            jax.ShapeDtypeStruct((NF, EMB), f32),
            jax.ShapeDtypeStruct((NF, H), f32),
            jax.ShapeDtypeStruct((NF, H), f32),
            jax.ShapeDtypeStruct((BS, EMB), f32),
        ),
    )(fe, wg, asw, adw, ne, w1a)


def _gat_sc_body(as_hbm, ad_hbm, se_hbm, de_hbm, s_o,
                 as_v, ad_v, se_v, de_v, s_v):
    wid = lax.axis_index("s") * 2 + lax.axis_index("c")
    head = wid & 1
    base = (wid >> 1) * EPW

    pltpu.sync_copy(as_hbm, as_v)
    pltpu.sync_copy(ad_hbm, ad_v)
    pltpu.sync_copy(se_hbm.at[pl.ds(base, EPW)], se_v)
    pltpu.sync_copy(de_hbm.at[pl.ds(base, EPW)], de_v)

    z16 = jnp.zeros((16,), f32)

    def _zero(i, _):
        off = pl.multiple_of(i * 128, 128)
        for u in range(8):
            s_v[pl.ds(off + u * 16, 16)] = z16
        return 0
    lax.fori_loop(0, (NF * NF) // 128, _zero, 0)

    def _edges(g, _):
        off = pl.multiple_of(g * 16, 16)
        sv = se_v[pl.ds(off, 16)]
        dv = de_v[pl.ds(off, 16)]
        asg = plsc.load_gather(as_v, [sv * H + head])
        adg = plsc.load_gather(ad_v, [dv * H + head])
        al = asg + adg
        al = jnp.where(al > 0, al, al * 0.2)
        ex = jnp.exp(al)
        plsc.addupdate_scatter(s_v, [dv * NF + sv], ex)
        return 0
    lax.fori_loop(0, EPW // 16, _edges, 0)

    pltpu.sync_copy(s_v, s_o.at[wid])


def _gat_sc(a_src, a_dst, se, de):
    mesh = plsc.VectorSubcoreMesh(core_axis_name="c", subcore_axis_name="s")
    fn = pl.kernel(
        _gat_sc_body, mesh=mesh,
        out_type=jax.ShapeDtypeStruct((NW, NF * NF), f32),
        scratch_types=[
            pltpu.VMEM((NF * H,), f32),
            pltpu.VMEM((NF * H,), f32),
            pltpu.VMEM((EPW,), jnp.int32),
            pltpu.VMEM((EPW,), jnp.int32),
            pltpu.VMEM((NF * NF,), f32),
        ],
        compiler_params=pltpu.CompilerParams(needs_layout_passes=False),
    )
    return fn(a_src.reshape(NF * H), a_dst.reshape(NF * H), se, de)


def _lrelu(z, s):
    return jnp.maximum(z, z * s)


def _fold_prep(a_r, sp_r, x_r, bias_r, w1b_r, g1_r, be1_r, at4_s, bt4_s):
    s0 = sp_r[0]
    s1m = sp_r[1]
    for w in range(2, NW, 2):
        s0 = s0 + sp_r[w]
        s1m = s1m + sp_r[w + 1]
    den0 = jnp.sum(s0, axis=1, keepdims=True) + 1e-16
    den1 = jnp.sum(s1m, axis=1, keepdims=True) + 1e-16
    x = x_r[...]
    agg0 = jnp.dot(s0, x[:, :32], preferred_element_type=f32) / den0
    agg1 = jnp.dot(s1m, x[:, 32:], preferred_element_type=f32) / den1
    msg = jnp.concatenate([agg0, agg1], axis=1) + bias_r[...]
    b = jnp.dot(msg, w1b_r[...], preferred_element_type=f32)
    a = a_r[...]
    am = jnp.mean(a, axis=0, keepdims=True)
    av = jnp.mean(a * a, axis=0, keepdims=True) - am * am
    bm = jnp.mean(b, axis=0, keepdims=True)
    bv = jnp.mean(b * b, axis=0, keepdims=True) - bm * bm
    sc1 = jax.lax.rsqrt(av + bv + 1e-5) * g1_r[...]
    at = (a - am) * sc1
    bt = (b - bm) * sc1 + be1_r[...]
    at4_s[...] = jnp.concatenate([at, at, at, at], axis=1)
    bt4_s[...] = jnp.concatenate(
        [bt[0:64], bt[64:128], bt[128:192], bt[192:256]], axis=1)


def _m1_body(a_r, sp_r, x_r, bias_r, w1b_r, g1_r, be1_r, w2u_r,
             stats_o, at4_o, bt4_o, at4_s, bt4_s):
    i = pl.program_id(0)

    @pl.when(i == 0)
    def _():
        _fold_prep(a_r, sp_r, x_r, bias_r, w1b_r, g1_r, be1_r, at4_s, bt4_s)
        stats_o[...] = jnp.zeros((2, 128), f32)

    ablk = at4_s[pl.ds(i * SB, SB), :]
    z = ablk[:, None, :] + bt4_s[...][None, :, :]
    h1 = _lrelu(z, 0.01).reshape(SB * EMB, 4 * EMB)
    y = jnp.dot(h1, w2u_r[...], preferred_element_type=f32)
    sy = jnp.sum(y, axis=0, keepdims=True)
    syy = jnp.sum(y * y, axis=0, keepdims=True)
    stats_o[...] = stats_o[...] + jnp.concatenate([sy, syy], axis=0)
    at4_o[...] = ablk
    bt4_o[...] = bt4_s[...]


def _m1(a, sparts, x, bias_gat, w1b, g1, be1, w2u):
    full = lambda s: pl.BlockSpec(s, lambda i: tuple(0 for _ in s))
    return pl.pallas_call(
        _m1_body,
        grid=(NBLK,),
        in_specs=[
            full((BS, EMB)),
            full((NW, NF, NF)),
            full((NF, EMB)),
            full((1, EMB)),
            full((EMB, EMB)),
            full((1, EMB)),
            full((1, EMB)),
            full((4 * EMB, 128)),
        ],
        out_specs=[
            full((2, 128)),
            pl.BlockSpec((SB, 4 * EMB), lambda i: (i, 0)),
            full((EMB, 4 * EMB)),
        ],
        out_shape=(
            jax.ShapeDtypeStruct((2, 128), f32),
            jax.ShapeDtypeStruct((BS, 4 * EMB), f32),
            jax.ShapeDtypeStruct((EMB, 4 * EMB), f32),
        ),
        scratch_shapes=[
            pltpu.VMEM((BS, 4 * EMB), f32),
            pltpu.VMEM((EMB, 4 * EMB), f32),
        ],
        compiler_params=pltpu.CompilerParams(
            dimension_semantics=("arbitrary",)),
    )(a, sparts, x, bias_gat, w1b, g1, be1, w2u)


def _m2_body(at4_r, bt4_r, w2s_r, c2_r, w3b_r, b3_r, out_o):
    z = at4_r[...][:, None, :] + bt4_r[...][None, :, :]
    h1 = _lrelu(z, 0.01).reshape(SB * EMB, 4 * EMB)
    yp = jnp.dot(h1, w2s_r[...], preferred_element_type=f32) + c2_r[...]
    h2 = _lrelu(yp, 0.01)
    v = jnp.dot(h2, w3b_r[...], preferred_element_type=f32) + b3_r[0, 0]
    v3 = v.reshape(SB, EMB, 4)
    out_o[...] = jnp.swapaxes(v3, 1, 2).reshape(SB, NF)


def _m2(at4, bt4, w2s, c2, w3b, b3):
    full = lambda s: pl.BlockSpec(s, lambda i: tuple(0 for _ in s))
    return pl.pallas_call(
        _m2_body,
        grid=(NBLK,),
        in_specs=[
            pl.BlockSpec((SB, 4 * EMB), lambda i: (i, 0)),
            full((EMB, 4 * EMB)),
            full((4 * EMB, 128)),
            full((1, 128)),
            full((128, 4)),
            full((1, 1)),
        ],
        out_specs=pl.BlockSpec((SB, NF), lambda i: (i, 0)),
        out_shape=jax.ShapeDtypeStruct((BS, NF), f32),
        compiler_params=pltpu.CompilerParams(
            dimension_semantics=("arbitrary",)),
    )(at4, bt4, w2s, c2, w3b, b3)


def kernel(node_emb, feature_emb, relation_index, W_gat, att_src, att_dst,
           bias_gat, W1, b1, g1, be1, W2, b2, g2, be2, W3, b3):
    se = relation_index[0].astype(jnp.int32)
    de = relation_index[1].astype(jnp.int32)
    asw = jnp.zeros((EMB, H), f32).at[:32, 0].set(att_src[0]).at[32:, 1].set(att_src[1])
    adw = jnp.zeros((EMB, H), f32).at[:32, 0].set(att_dst[0]).at[32:, 1].set(att_dst[1])
    w1a = W1[:EMB]
    w1b = W1[EMB:]
    zpad = jnp.zeros((EMB, 32), f32)
    w2u = jnp.concatenate([
        jnp.concatenate([W2, zpad, zpad, zpad], axis=1),
        jnp.concatenate([zpad, W2, zpad, zpad], axis=1),
        jnp.concatenate([zpad, zpad, W2, zpad], axis=1),
        jnp.concatenate([zpad, zpad, zpad, W2], axis=1)], axis=0)

    x, a_src, a_dst, a = _prep(feature_emb, W_gat, asw, adw, node_emb, w1a)
    sparts = _gat_sc(a_src, a_dst, se, de)

    stats, at4, bt4 = _m1(
        a, sparts.reshape(NW, NF, NF), x, bias_gat.reshape(1, EMB), w1b,
        g1.reshape(1, EMB), be1.reshape(1, EMB), w2u)

    sy = (stats[0, 0:32] + stats[0, 32:64] + stats[0, 64:96] + stats[0, 96:128])
    syy = (stats[1, 0:32] + stats[1, 32:64] + stats[1, 64:96] + stats[1, 96:128])
    my = sy / NPAIR
    vy = syy / NPAIR - my * my
    s2 = g2 * jax.lax.rsqrt(vy + 1e-5)
    w2s = w2u * jnp.tile(s2, 4)[None, :]
    c2 = jnp.tile(be2 - my * s2, 4).reshape(1, 128)
    w3col = W3[:, 0]
    z32 = jnp.zeros((32,), f32)
    w3b = jnp.stack([
        jnp.concatenate([w3col, z32, z32, z32]),
        jnp.concatenate([z32, w3col, z32, z32]),
        jnp.concatenate([z32, z32, w3col, z32]),
        jnp.concatenate([z32, z32, z32, w3col])], axis=1)

    out = _m2(at4, bt4, w2s, c2, w3b, b3.reshape(1, 1))
    return out


# speedup vs baseline: 36.0449x; 1.1123x over previous
import jax
import jax.numpy as jnp
from jax import lax
from jax.experimental import pallas as pl
from jax.experimental.pallas import tpu as pltpu
from jax.experimental.pallas import tpu_sc as plsc

BS = 2048
NF = 256
EMB = 64
H = 2
E = 8192
NWK = 8
EPW = E // (NWK // H)
SB = 64
NBLK = BS // SB
NPAIR = BS * NF

f32 = jnp.float32


def _prep_body(fe, wg, asw, adw, ne, w1a, x_o, as_o, ad_o, a_o):
    x = jnp.dot(fe[...], wg[...], preferred_element_type=f32)
    x_o[...] = x
    as_o[...] = jnp.dot(x, asw[...], preferred_element_type=f32)
    ad_o[...] = jnp.dot(x, adw[...], preferred_element_type=f32)
    a_o[...] = jnp.dot(ne[...], w1a[...], preferred_element_type=f32)


def _prep(fe, wg, asw, adw, ne, w1a):
    return pl.pallas_call(
        _prep_body,
        out_shape=(
            jax.ShapeDtypeStruct((NF, EMB), f32),
            jax.ShapeDtypeStruct((NF, H), f32),
            jax.ShapeDtypeStruct((NF, H), f32),
            jax.ShapeDtypeStruct((BS, EMB), f32),
        ),
    )(fe, wg, asw, adw, ne, w1a)


def _gat_sc_body(as_hbm, ad_hbm, se_hbm, de_hbm, s_o,
                 as_v, ad_v, se_v, de_v, s_v):
    wid = lax.axis_index("s") * 2 + lax.axis_index("c")

    @pl.when(wid < NWK)
    def _():
        head = wid & 1
        base = (wid >> 1) * EPW

        pltpu.sync_copy(as_hbm, as_v)
        pltpu.sync_copy(ad_hbm, ad_v)
        pltpu.sync_copy(se_hbm.at[pl.ds(base, EPW)], se_v)
        pltpu.sync_copy(de_hbm.at[pl.ds(base, EPW)], de_v)

        z16 = jnp.zeros((16,), f32)

        def _zero(r, _):
            for u in range(NF // 16):
                s_v[r, pl.ds(u * 16, 16)] = z16
            return 0
        lax.fori_loop(0, NF, _zero, 0)

        def _edges(g, _):
            off = pl.multiple_of(g * 16, 16)
            sv = se_v[pl.ds(off, 16)]
            dv = de_v[pl.ds(off, 16)]
            asg = plsc.load_gather(as_v, [sv * H + head])
            adg = plsc.load_gather(ad_v, [dv * H + head])
            al = asg + adg
            al = jnp.where(al > 0, al, al * 0.2)
            ex = jnp.exp(al)
            plsc.addupdate_scatter(s_v, [dv, sv], ex)
            return 0
        lax.fori_loop(0, EPW // 16, _edges, 0)

        pltpu.sync_copy(s_v, s_o.at[wid])


def _gat_sc(a_src, a_dst, se, de):
    mesh = plsc.VectorSubcoreMesh(core_axis_name="c", subcore_axis_name="s")
    fn = pl.kernel(
        _gat_sc_body, mesh=mesh,
        out_type=jax.ShapeDtypeStruct((NWK, NF, NF), f32),
        scratch_types=[
            pltpu.VMEM((NF * H,), f32),
            pltpu.VMEM((NF * H,), f32),
            pltpu.VMEM((EPW,), jnp.int32),
            pltpu.VMEM((EPW,), jnp.int32),
            pltpu.VMEM((NF, NF), f32),
        ],
        compiler_params=pltpu.CompilerParams(needs_layout_passes=False),
    )
    return fn(a_src.reshape(NF * H), a_dst.reshape(NF * H), se, de)


def _lrelu(z, s):
    return jnp.maximum(z, z * s)


def _fold_prep(a_r, sp_r, x_r, bias_r, w1b_r, g1_r, be1_r, at4_s, bt4_s):
    s0 = sp_r[0]
    s1m = sp_r[1]
    for w in range(2, NWK, 2):
        s0 = s0 + sp_r[w]
        s1m = s1m + sp_r[w + 1]
    den0 = jnp.sum(s0, axis=1, keepdims=True) + 1e-16
    den1 = jnp.sum(s1m, axis=1, keepdims=True) + 1e-16
    x = x_r[...]
    agg0 = jnp.dot(s0, x[:, :32], preferred_element_type=f32) / den0
    agg1 = jnp.dot(s1m, x[:, 32:], preferred_element_type=f32) / den1
    msg = jnp.concatenate([agg0, agg1], axis=1) + bias_r[...]
    b = jnp.dot(msg, w1b_r[...], preferred_element_type=f32)
    a = a_r[...]
    am = jnp.mean(a, axis=0, keepdims=True)
    av = jnp.mean(a * a, axis=0, keepdims=True) - am * am
    bm = jnp.mean(b, axis=0, keepdims=True)
    bv = jnp.mean(b * b, axis=0, keepdims=True) - bm * bm
    sc1 = jax.lax.rsqrt(av + bv + 1e-5) * g1_r[...]
    at = (a - am) * sc1
    bt = (b - bm) * sc1 + be1_r[...]
    at4_s[...] = jnp.concatenate([at, at, at, at], axis=1)
    bt4_s[...] = jnp.concatenate(
        [bt[0:64], bt[64:128], bt[128:192], bt[192:256]], axis=1)


def _mlp_body(a_r, sp_r, x_r, bias_r, w1b_r, g1_r, be1_r, w2u_r, g2_r, be2_r,
              w3b_r, b3_r, out_o, at4_s, bt4_s, stats_s, w2s_s, c2_s):
    i = pl.program_id(0)

    @pl.when(i == 0)
    def _():
        _fold_prep(a_r, sp_r, x_r, bias_r, w1b_r, g1_r, be1_r, at4_s, bt4_s)
        stats_s[...] = jnp.zeros((2, 128), f32)

    @pl.when(i < NBLK)
    def _():
        ablk = at4_s[pl.ds(i * SB, SB), :]
        z = ablk[:, None, :] + bt4_s[...][None, :, :]
        h1 = _lrelu(z, 0.01).reshape(SB * EMB, 4 * EMB)
        y = jnp.dot(h1, w2u_r[...], preferred_element_type=f32)
        sy = jnp.sum(y, axis=0, keepdims=True)
        syy = jnp.sum(y * y, axis=0, keepdims=True)
        stats_s[...] = stats_s[...] + jnp.concatenate([sy, syy], axis=0)

    @pl.when(i == NBLK)
    def _():
        st = stats_s[...]
        sy = (st[0:1, 0:32] + st[0:1, 32:64] + st[0:1, 64:96]
              + st[0:1, 96:128])
        syy = (st[1:2, 0:32] + st[1:2, 32:64] + st[1:2, 64:96]
               + st[1:2, 96:128])
        my = sy / NPAIR
        vy = syy / NPAIR - my * my
        s2 = g2_r[...] * jax.lax.rsqrt(vy + 1e-5)
        c2 = be2_r[...] - my * s2
        s2t = jnp.concatenate([s2, s2, s2, s2], axis=1)
        w2s_s[...] = w2u_r[...] * s2t
        c2_s[...] = jnp.concatenate([c2, c2, c2, c2], axis=1)

    @pl.when(i >= NBLK)
    def _():
        j = i - NBLK
        ablk = at4_s[pl.ds(j * SB, SB), :]
        z = ablk[:, None, :] + bt4_s[...][None, :, :]
        h1 = _lrelu(z, 0.01).reshape(SB * EMB, 4 * EMB)
        yp = jnp.dot(h1, w2s_s[...], preferred_element_type=f32) + c2_s[...]
        h2 = _lrelu(yp, 0.01)
        v = jnp.dot(h2, w3b_r[...], preferred_element_type=f32) + b3_r[0, 0]
        v3 = v.reshape(SB, EMB, 4)
        out_o[...] = jnp.swapaxes(v3, 1, 2).reshape(SB, NF)


def _mlp(a, sparts, x, bias_gat, w1b, g1, be1, w2u, g2, be2, w3b, b3):
    full = lambda s: pl.BlockSpec(s, lambda i: tuple(0 for _ in s))
    return pl.pallas_call(
        _mlp_body,
        grid=(2 * NBLK,),
        in_specs=[
            full((BS, EMB)),
            full((NWK, NF, NF)),
            full((NF, EMB)),
            full((1, EMB)),
            full((EMB, EMB)),
            full((1, EMB)),
            full((1, EMB)),
            full((4 * EMB, 128)),
            full((1, 32)),
            full((1, 32)),
            full((128, 4)),
            full((1, 1)),
        ],
        out_specs=pl.BlockSpec(
            (SB, NF), lambda i: (jnp.maximum(i - NBLK, 0), 0)),
        out_shape=jax.ShapeDtypeStruct((BS, NF), f32),
        scratch_shapes=[
            pltpu.VMEM((BS, 4 * EMB), f32),
            pltpu.VMEM((EMB, 4 * EMB), f32),
            pltpu.VMEM((2, 128), f32),
            pltpu.VMEM((4 * EMB, 128), f32),
            pltpu.VMEM((1, 128), f32),
        ],
        compiler_params=pltpu.CompilerParams(
            dimension_semantics=("arbitrary",)),
    )(a, sparts, x, bias_gat, w1b, g1, be1, w2u, g2, be2, w3b, b3)


def kernel(node_emb, feature_emb, relation_index, W_gat, att_src, att_dst,
           bias_gat, W1, b1, g1, be1, W2, b2, g2, be2, W3, b3):
    se = relation_index[0].astype(jnp.int32)
    de = relation_index[1].astype(jnp.int32)
    asw = jnp.zeros((EMB, H), f32).at[:32, 0].set(att_src[0]).at[32:, 1].set(att_src[1])
    adw = jnp.zeros((EMB, H), f32).at[:32, 0].set(att_dst[0]).at[32:, 1].set(att_dst[1])
    w1a = W1[:EMB]
    w1b = W1[EMB:]
    zpad = jnp.zeros((EMB, 32), f32)
    w2u = jnp.concatenate([
        jnp.concatenate([W2, zpad, zpad, zpad], axis=1),
        jnp.concatenate([zpad, W2, zpad, zpad], axis=1),
        jnp.concatenate([zpad, zpad, W2, zpad], axis=1),
        jnp.concatenate([zpad, zpad, zpad, W2], axis=1)], axis=0)
    w3col = W3[:, 0]
    z32 = jnp.zeros((32,), f32)
    w3b = jnp.stack([
        jnp.concatenate([w3col, z32, z32, z32]),
        jnp.concatenate([z32, w3col, z32, z32]),
        jnp.concatenate([z32, z32, w3col, z32]),
        jnp.concatenate([z32, z32, z32, w3col])], axis=1)

    x, a_src, a_dst, a = _prep(feature_emb, W_gat, asw, adw, node_emb, w1a)
    sparts = _gat_sc(a_src, a_dst, se, de)

    out = _mlp(a, sparts, x, bias_gat.reshape(1, EMB), w1b,
               g1.reshape(1, EMB), be1.reshape(1, EMB), w2u,
               g2.reshape(1, 32), be2.reshape(1, 32), w3b, b3.reshape(1, 1))
    return out


# speedup vs baseline: 36.0701x; 1.0007x over previous
import jax
import jax.numpy as jnp
from jax import lax
from jax.experimental import pallas as pl
from jax.experimental.pallas import tpu as pltpu
from jax.experimental.pallas import tpu_sc as plsc

BS = 2048
NF = 256
EMB = 64
H = 2
E = 8192
NWK = 8
EPW = E // (NWK // H)
SB = 64
NBLK = BS // SB
NPAIR = BS * NF

f32 = jnp.float32


def _prep_body(fe, wg, asw, adw, ne, w1a, x_o, as_o, ad_o, a_o):
    x = jnp.dot(fe[...], wg[...], preferred_element_type=f32)
    x_o[...] = x
    as_o[...] = jnp.dot(x, asw[...], preferred_element_type=f32)
    ad_o[...] = jnp.dot(x, adw[...], preferred_element_type=f32)
    a_o[...] = jnp.dot(ne[...], w1a[...], preferred_element_type=f32)


def _prep(fe, wg, asw, adw, ne, w1a):
    return pl.pallas_call(
        _prep_body,
        out_shape=(
            jax.ShapeDtypeStruct((NF, EMB), f32),
            jax.ShapeDtypeStruct((NF, H), f32),
            jax.ShapeDtypeStruct((NF, H), f32),
            jax.ShapeDtypeStruct((BS, EMB), f32),
        ),
    )(fe, wg, asw, adw, ne, w1a)


def _gat_sc_body(as_hbm, ad_hbm, se_hbm, de_hbm, s_o,
                 as_v, ad_v, se_v, de_v, s_v):
    wid = lax.axis_index("s") * 2 + lax.axis_index("c")

    @pl.when(wid < NWK)
    def _():
        head = wid & 1
        base = (wid >> 1) * EPW

        pltpu.sync_copy(as_hbm, as_v)
        pltpu.sync_copy(ad_hbm, ad_v)
        pltpu.sync_copy(se_hbm.at[pl.ds(base, EPW)], se_v)
        pltpu.sync_copy(de_hbm.at[pl.ds(base, EPW)], de_v)

        z16 = jnp.zeros((16,), f32)

        def _zero(r, _):
            for u in range(NF // 16):
                s_v[r, pl.ds(u * 16, 16)] = z16
            return 0
        lax.fori_loop(0, NF, _zero, 0)

        def _edges(g, _):
            off = pl.multiple_of(g * 16, 16)
            sv = se_v[pl.ds(off, 16)]
            dv = de_v[pl.ds(off, 16)]
            asg = plsc.load_gather(as_v, [sv * H + head])
            adg = plsc.load_gather(ad_v, [dv * H + head])
            al = asg + adg
            al = jnp.where(al > 0, al, al * 0.2)
            ex = jnp.exp(al)
            plsc.addupdate_scatter(s_v, [dv, sv], ex)
            return 0
        lax.fori_loop(0, EPW // 16, _edges, 0)

        pltpu.sync_copy(s_v, s_o.at[wid])


def _gat_sc(a_src, a_dst, se, de):
    mesh = plsc.VectorSubcoreMesh(core_axis_name="c", subcore_axis_name="s")
    fn = pl.kernel(
        _gat_sc_body, mesh=mesh,
        out_type=jax.ShapeDtypeStruct((NWK, NF, NF), f32),
        scratch_types=[
            pltpu.VMEM((NF * H,), f32),
            pltpu.VMEM((NF * H,), f32),
            pltpu.VMEM((EPW,), jnp.int32),
            pltpu.VMEM((EPW,), jnp.int32),
            pltpu.VMEM((NF, NF), f32),
        ],
        compiler_params=pltpu.CompilerParams(needs_layout_passes=False),
    )
    return fn(a_src.reshape(NF * H), a_dst.reshape(NF * H), se, de)


def _lrelu(z, s):
    return jnp.maximum(z, z * s)


def _fold_prep(a_r, sp_r, x_r, bias_r, w1b_r, g1_r, be1_r, at4_s, bt4_s):
    s0 = sp_r[0]
    s1m = sp_r[1]
    for w in range(2, NWK, 2):
        s0 = s0 + sp_r[w]
        s1m = s1m + sp_r[w + 1]
    den0 = jnp.sum(s0, axis=1, keepdims=True) + 1e-16
    den1 = jnp.sum(s1m, axis=1, keepdims=True) + 1e-16
    x = x_r[...]
    agg0 = jnp.dot(s0, x[:, :32], preferred_element_type=f32) / den0
    agg1 = jnp.dot(s1m, x[:, 32:], preferred_element_type=f32) / den1
    msg = jnp.concatenate([agg0, agg1], axis=1) + bias_r[...]
    b = jnp.dot(msg, w1b_r[...], preferred_element_type=f32)
    a = a_r[...]
    am = jnp.mean(a, axis=0, keepdims=True)
    av = jnp.mean(a * a, axis=0, keepdims=True) - am * am
    bm = jnp.mean(b, axis=0, keepdims=True)
    bv = jnp.mean(b * b, axis=0, keepdims=True) - bm * bm
    sc1 = jax.lax.rsqrt(av + bv + 1e-5) * g1_r[...]
    at = (a - am) * sc1
    bt = (b - bm) * sc1 + be1_r[...]
    at4_s[...] = jnp.concatenate([at, at, at, at], axis=1).astype(jnp.bfloat16)
    bt4_s[...] = jnp.concatenate(
        [bt[0:64], bt[64:128], bt[128:192], bt[192:256]],
        axis=1).astype(jnp.bfloat16)


def _mlp_body(a_r, sp_r, x_r, bias_r, w1b_r, g1_r, be1_r, w2u_r, g2_r, be2_r,
              w3b_r, b3_r, out_o, at4_s, bt4_s, stats_s, w2s_s, c2_s):
    i = pl.program_id(0)

    @pl.when(i == 0)
    def _():
        _fold_prep(a_r, sp_r, x_r, bias_r, w1b_r, g1_r, be1_r, at4_s, bt4_s)
        stats_s[...] = jnp.zeros((2, 128), f32)

    @pl.when(i < NBLK)
    def _():
        ablk = at4_s[pl.ds(i * SB, SB), :]
        z = ablk[:, None, :] + bt4_s[...][None, :, :]
        h1 = _lrelu(z, 0.01).reshape(SB * EMB, 4 * EMB)
        y = jnp.dot(h1, w2u_r[...], preferred_element_type=f32)
        sy = jnp.sum(y, axis=0, keepdims=True)
        syy = jnp.sum(y * y, axis=0, keepdims=True)
        stats_s[...] = stats_s[...] + jnp.concatenate([sy, syy], axis=0)

    @pl.when(i == NBLK)
    def _():
        st = stats_s[...]
        sy = (st[0:1, 0:32] + st[0:1, 32:64] + st[0:1, 64:96]
              + st[0:1, 96:128])
        syy = (st[1:2, 0:32] + st[1:2, 32:64] + st[1:2, 64:96]
               + st[1:2, 96:128])
        my = sy / NPAIR
        vy = syy / NPAIR - my * my
        s2 = g2_r[...] * jax.lax.rsqrt(vy + 1e-5)
        c2 = be2_r[...] - my * s2
        s2t = jnp.concatenate([s2, s2, s2, s2], axis=1)
        w2s_s[...] = (w2u_r[...].astype(f32) * s2t).astype(jnp.bfloat16)
        c2_s[...] = jnp.concatenate([c2, c2, c2, c2], axis=1)

    @pl.when(i >= NBLK)
    def _():
        j = i - NBLK
        ablk = at4_s[pl.ds(j * SB, SB), :]
        z = ablk[:, None, :] + bt4_s[...][None, :, :]
        h1 = _lrelu(z, 0.01).reshape(SB * EMB, 4 * EMB)
        yp = jnp.dot(h1, w2s_s[...], preferred_element_type=f32) + c2_s[...]
        h2 = _lrelu(yp, 0.01)
        v = jnp.dot(h2, w3b_r[...], preferred_element_type=f32) + b3_r[0, 0]
        v3 = v.reshape(SB, EMB, 4)
        out_o[...] = jnp.swapaxes(v3, 1, 2).reshape(SB, NF)


def _mlp(a, sparts, x, bias_gat, w1b, g1, be1, w2u, g2, be2, w3b, b3):
    full = lambda s: pl.BlockSpec(s, lambda i: tuple(0 for _ in s))
    return pl.pallas_call(
        _mlp_body,
        grid=(2 * NBLK,),
        in_specs=[
            full((BS, EMB)),
            full((NWK, NF, NF)),
            full((NF, EMB)),
            full((1, EMB)),
            full((EMB, EMB)),
            full((1, EMB)),
            full((1, EMB)),
            full((4 * EMB, 128)),
            full((1, 32)),
            full((1, 32)),
            full((128, 4)),
            full((1, 1)),
        ],
        out_specs=pl.BlockSpec(
            (SB, NF), lambda i: (jnp.maximum(i - NBLK, 0), 0)),
        out_shape=jax.ShapeDtypeStruct((BS, NF), f32),
        scratch_shapes=[
            pltpu.VMEM((BS, 4 * EMB), jnp.bfloat16),
            pltpu.VMEM((EMB, 4 * EMB), jnp.bfloat16),
            pltpu.VMEM((2, 128), f32),
            pltpu.VMEM((4 * EMB, 128), jnp.bfloat16),
            pltpu.VMEM((1, 128), f32),
        ],
        compiler_params=pltpu.CompilerParams(
            dimension_semantics=("arbitrary",)),
    )(a, sparts, x, bias_gat, w1b, g1, be1, w2u, g2, be2, w3b, b3)


def kernel(node_emb, feature_emb, relation_index, W_gat, att_src, att_dst,
           bias_gat, W1, b1, g1, be1, W2, b2, g2, be2, W3, b3):
    se = relation_index[0].astype(jnp.int32)
    de = relation_index[1].astype(jnp.int32)
    asw = jnp.zeros((EMB, H), f32).at[:32, 0].set(att_src[0]).at[32:, 1].set(att_src[1])
    adw = jnp.zeros((EMB, H), f32).at[:32, 0].set(att_dst[0]).at[32:, 1].set(att_dst[1])
    w1a = W1[:EMB]
    w1b = W1[EMB:]
    zpad = jnp.zeros((EMB, 32), f32)
    w2u = jnp.concatenate([
        jnp.concatenate([W2, zpad, zpad, zpad], axis=1),
        jnp.concatenate([zpad, W2, zpad, zpad], axis=1),
        jnp.concatenate([zpad, zpad, W2, zpad], axis=1),
        jnp.concatenate([zpad, zpad, zpad, W2], axis=1)],
        axis=0).astype(jnp.bfloat16)
    w3col = W3[:, 0]
    z32 = jnp.zeros((32,), f32)
    w3b = jnp.stack([
        jnp.concatenate([w3col, z32, z32, z32]),
        jnp.concatenate([z32, w3col, z32, z32]),
        jnp.concatenate([z32, z32, w3col, z32]),
        jnp.concatenate([z32, z32, z32, w3col])], axis=1)

    x, a_src, a_dst, a = _prep(feature_emb, W_gat, asw, adw, node_emb, w1a)
    sparts = _gat_sc(a_src, a_dst, se, de)

    out = _mlp(a, sparts, x, bias_gat.reshape(1, EMB), w1b,
               g1.reshape(1, EMB), be1.reshape(1, EMB), w2u,
               g2.reshape(1, 32), be2.reshape(1, 32), w3b, b3.reshape(1, 1))
    return out


# speedup vs baseline: 38.1459x; 1.0575x over previous
import jax
import jax.numpy as jnp
from jax import lax
from jax.experimental import pallas as pl
from jax.experimental.pallas import tpu as pltpu
from jax.experimental.pallas import tpu_sc as plsc

BS = 2048
NF = 256
EMB = 64
H = 2
E = 8192
NWK = 8
EPW = E // (NWK // H)
SB = 128
NBLK = BS // SB
NPAIR = BS * NF

f32 = jnp.float32


def _prep_body(fe, wg, asw, adw, ne, w1a, x_o, as_o, ad_o, a_o):
    x = jnp.dot(fe[...], wg[...], preferred_element_type=f32)
    x_o[...] = x
    as_o[...] = jnp.dot(x, asw[...], preferred_element_type=f32)
    ad_o[...] = jnp.dot(x, adw[...], preferred_element_type=f32)
    a_o[...] = jnp.dot(ne[...], w1a[...], preferred_element_type=f32)


def _prep(fe, wg, asw, adw, ne, w1a):
    return pl.pallas_call(
        _prep_body,
        out_shape=(
            jax.ShapeDtypeStruct((NF, EMB), f32),
            jax.ShapeDtypeStruct((NF, H), f32),
            jax.ShapeDtypeStruct((NF, H), f32),
            jax.ShapeDtypeStruct((BS, EMB), f32),
        ),
    )(fe, wg, asw, adw, ne, w1a)


def _gat_sc_body(as_hbm, ad_hbm, se_hbm, de_hbm, s_o,
                 as_v, ad_v, se_v, de_v, s_v):
    wid = lax.axis_index("s") * 2 + lax.axis_index("c")

    @pl.when(wid < NWK)
    def _():
        head = wid & 1
        base = (wid >> 1) * EPW

        pltpu.sync_copy(as_hbm, as_v)
        pltpu.sync_copy(ad_hbm, ad_v)
        pltpu.sync_copy(se_hbm.at[pl.ds(base, EPW)], se_v)
        pltpu.sync_copy(de_hbm.at[pl.ds(base, EPW)], de_v)

        z16 = jnp.zeros((16,), f32)

        def _zero(r, _):
            for u in range(NF // 16):
                s_v[r, pl.ds(u * 16, 16)] = z16
            return 0
        lax.fori_loop(0, NF, _zero, 0)

        def _edges(g, _):
            off = pl.multiple_of(g * 16, 16)
            sv = se_v[pl.ds(off, 16)]
            dv = de_v[pl.ds(off, 16)]
            asg = plsc.load_gather(as_v, [sv * H + head])
            adg = plsc.load_gather(ad_v, [dv * H + head])
            al = asg + adg
            al = jnp.where(al > 0, al, al * 0.2)
            ex = jnp.exp(al)
            plsc.addupdate_scatter(s_v, [dv, sv], ex)
            return 0
        lax.fori_loop(0, EPW // 16, _edges, 0)

        pltpu.sync_copy(s_v, s_o.at[wid])


def _gat_sc(a_src, a_dst, se, de):
    mesh = plsc.VectorSubcoreMesh(core_axis_name="c", subcore_axis_name="s")
    fn = pl.kernel(
        _gat_sc_body, mesh=mesh,
        out_type=jax.ShapeDtypeStruct((NWK, NF, NF), f32),
        scratch_types=[
            pltpu.VMEM((NF * H,), f32),
            pltpu.VMEM((NF * H,), f32),
            pltpu.VMEM((EPW,), jnp.int32),
            pltpu.VMEM((EPW,), jnp.int32),
            pltpu.VMEM((NF, NF), f32),
        ],
        compiler_params=pltpu.CompilerParams(needs_layout_passes=False),
    )
    return fn(a_src.reshape(NF * H), a_dst.reshape(NF * H), se, de)


def _lrelu(z, s):
    return jnp.maximum(z, z * s)


def _fold_prep(a_r, sp_r, x_r, bias_r, w1b_r, g1_r, be1_r, at4_s, bt4_s):
    s0 = sp_r[0]
    s1m = sp_r[1]
    for w in range(2, NWK, 2):
        s0 = s0 + sp_r[w]
        s1m = s1m + sp_r[w + 1]
    den0 = jnp.sum(s0, axis=1, keepdims=True) + 1e-16
    den1 = jnp.sum(s1m, axis=1, keepdims=True) + 1e-16
    x = x_r[...]
    agg0 = jnp.dot(s0, x[:, :32], preferred_element_type=f32) / den0
    agg1 = jnp.dot(s1m, x[:, 32:], preferred_element_type=f32) / den1
    msg = jnp.concatenate([agg0, agg1], axis=1) + bias_r[...]
    b = jnp.dot(msg, w1b_r[...], preferred_element_type=f32)
    a = a_r[...]
    am = jnp.mean(a, axis=0, keepdims=True)
    av = jnp.mean(a * a, axis=0, keepdims=True) - am * am
    bm = jnp.mean(b, axis=0, keepdims=True)
    bv = jnp.mean(b * b, axis=0, keepdims=True) - bm * bm
    sc1 = jax.lax.rsqrt(av + bv + 1e-5) * g1_r[...]
    at = (a - am) * sc1
    bt = (b - bm) * sc1 + be1_r[...]
    at4_s[...] = jnp.concatenate([at, at, at, at], axis=1).astype(jnp.bfloat16)
    bt4_s[...] = jnp.concatenate(
        [bt[0:64], bt[64:128], bt[128:192], bt[192:256]],
        axis=1).astype(jnp.bfloat16)


def _mlp_body(a_r, sp_r, x_r, bias_r, w1b_r, g1_r, be1_r, w2u_r, g2_r, be2_r,
              w3b_r, b3_r, out_o, at4_s, bt4_s, stats_s, w2s_s, c2_s):
    i = pl.program_id(0)

    @pl.when(i == 0)
    def _():
        _fold_prep(a_r, sp_r, x_r, bias_r, w1b_r, g1_r, be1_r, at4_s, bt4_s)
        stats_s[...] = jnp.zeros((2, 128), f32)

    @pl.when(i < NBLK)
    def _():
        ablk = at4_s[pl.ds(i * SB, SB), :]
        z = ablk[:, None, :] + bt4_s[...][None, :, :]
        h1 = _lrelu(z, 0.01).reshape(SB * EMB, 4 * EMB)
        y = jnp.dot(h1, w2u_r[...], preferred_element_type=f32)
        sy = jnp.sum(y, axis=0, keepdims=True)
        syy = jnp.sum(y * y, axis=0, keepdims=True)
        stats_s[...] = stats_s[...] + jnp.concatenate([sy, syy], axis=0)

    @pl.when(i == NBLK)
    def _():
        st = stats_s[...]
        sy = (st[0:1, 0:32] + st[0:1, 32:64] + st[0:1, 64:96]
              + st[0:1, 96:128])
        syy = (st[1:2, 0:32] + st[1:2, 32:64] + st[1:2, 64:96]
               + st[1:2, 96:128])
        my = sy / NPAIR
        vy = syy / NPAIR - my * my
        s2 = g2_r[...] * jax.lax.rsqrt(vy + 1e-5)
        c2 = be2_r[...] - my * s2
        s2t = jnp.concatenate([s2, s2, s2, s2], axis=1)
        w2s_s[...] = (w2u_r[...].astype(f32) * s2t).astype(jnp.bfloat16)
        c2_s[...] = jnp.concatenate([c2, c2, c2, c2], axis=1)

    @pl.when(i >= NBLK)
    def _():
        j = i - NBLK
        ablk = at4_s[pl.ds(j * SB, SB), :]
        z = ablk[:, None, :] + bt4_s[...][None, :, :]
        h1 = _lrelu(z, 0.01).reshape(SB * EMB, 4 * EMB)
        yp = jnp.dot(h1, w2s_s[...], preferred_element_type=f32) + c2_s[...]
        h2 = _lrelu(yp, 0.01)
        v = jnp.dot(h2, w3b_r[...], preferred_element_type=f32) + b3_r[0, 0]
        v3 = v.reshape(SB, EMB, 4)
        out_o[...] = jnp.swapaxes(v3, 1, 2).reshape(SB, NF)


def _mlp(a, sparts, x, bias_gat, w1b, g1, be1, w2u, g2, be2, w3b, b3):
    full = lambda s: pl.BlockSpec(s, lambda i: tuple(0 for _ in s))
    return pl.pallas_call(
        _mlp_body,
        grid=(2 * NBLK,),
        in_specs=[
            full((BS, EMB)),
            full((NWK, NF, NF)),
            full((NF, EMB)),
            full((1, EMB)),
            full((EMB, EMB)),
            full((1, EMB)),
            full((1, EMB)),
            full((4 * EMB, 128)),
            full((1, 32)),
            full((1, 32)),
            full((128, 4)),
            full((1, 1)),
        ],
        out_specs=pl.BlockSpec(
            (SB, NF), lambda i: (jnp.maximum(i - NBLK, 0), 0)),
        out_shape=jax.ShapeDtypeStruct((BS, NF), f32),
        scratch_shapes=[
            pltpu.VMEM((BS, 4 * EMB), jnp.bfloat16),
            pltpu.VMEM((EMB, 4 * EMB), jnp.bfloat16),
            pltpu.VMEM((2, 128), f32),
            pltpu.VMEM((4 * EMB, 128), jnp.bfloat16),
            pltpu.VMEM((1, 128), f32),
        ],
        compiler_params=pltpu.CompilerParams(
            dimension_semantics=("arbitrary",)),
    )(a, sparts, x, bias_gat, w1b, g1, be1, w2u, g2, be2, w3b, b3)


def kernel(node_emb, feature_emb, relation_index, W_gat, att_src, att_dst,
           bias_gat, W1, b1, g1, be1, W2, b2, g2, be2, W3, b3):
    se = relation_index[0].astype(jnp.int32)
    de = relation_index[1].astype(jnp.int32)
    asw = jnp.zeros((EMB, H), f32).at[:32, 0].set(att_src[0]).at[32:, 1].set(att_src[1])
    adw = jnp.zeros((EMB, H), f32).at[:32, 0].set(att_dst[0]).at[32:, 1].set(att_dst[1])
    w1a = W1[:EMB]
    w1b = W1[EMB:]
    zpad = jnp.zeros((EMB, 32), f32)
    w2u = jnp.concatenate([
        jnp.concatenate([W2, zpad, zpad, zpad], axis=1),
        jnp.concatenate([zpad, W2, zpad, zpad], axis=1),
        jnp.concatenate([zpad, zpad, W2, zpad], axis=1),
        jnp.concatenate([zpad, zpad, zpad, W2], axis=1)],
        axis=0).astype(jnp.bfloat16)
    w3col = W3[:, 0]
    z32 = jnp.zeros((32,), f32)
    w3b = jnp.stack([
        jnp.concatenate([w3col, z32, z32, z32]),
        jnp.concatenate([z32, w3col, z32, z32]),
        jnp.concatenate([z32, z32, w3col, z32]),
        jnp.concatenate([z32, z32, z32, w3col])], axis=1)

    x, a_src, a_dst, a = _prep(feature_emb, W_gat, asw, adw, node_emb, w1a)
    sparts = _gat_sc(a_src, a_dst, se, de)

    out = _mlp(a, sparts, x, bias_gat.reshape(1, EMB), w1b,
               g1.reshape(1, EMB), be1.reshape(1, EMB), w2u,
               g2.reshape(1, 32), be2.reshape(1, 32), w3b, b3.reshape(1, 1))
    return out


# speedup vs baseline: 40.4518x; 1.0604x over previous
import jax
import jax.numpy as jnp
from jax import lax
from jax.experimental import pallas as pl
from jax.experimental.pallas import tpu as pltpu
from jax.experimental.pallas import tpu_sc as plsc

BS = 2048
NF = 256
EMB = 64
H = 2
E = 8192
NWK = 8
EPW = E // (NWK // H)
SB = 256
NBLK = BS // SB
NPAIR = BS * NF

f32 = jnp.float32


def _prep_body(fe, asw2, adw2, as_o, ad_o):
    f = fe[...]
    as_o[...] = jnp.dot(f, asw2[...], preferred_element_type=f32)
    ad_o[...] = jnp.dot(f, adw2[...], preferred_element_type=f32)


def _prep(fe, asw2, adw2):
    return pl.pallas_call(
        _prep_body,
        out_shape=(
            jax.ShapeDtypeStruct((NF, H), f32),
            jax.ShapeDtypeStruct((NF, H), f32),
        ),
    )(fe, asw2, adw2)


def _gat_sc_body(as_hbm, ad_hbm, se_hbm, de_hbm, s_o,
                 as_v, ad_v, se_v, de_v, s_v):
    wid = lax.axis_index("s") * 2 + lax.axis_index("c")

    @pl.when(wid < NWK)
    def _():
        head = wid & 1
        base = (wid >> 1) * EPW

        pltpu.sync_copy(as_hbm, as_v)
        pltpu.sync_copy(ad_hbm, ad_v)
        pltpu.sync_copy(se_hbm.at[pl.ds(base, EPW)], se_v)
        pltpu.sync_copy(de_hbm.at[pl.ds(base, EPW)], de_v)

        z16 = jnp.zeros((16,), f32)

        def _zero(r, _):
            for u in range(NF // 16):
                s_v[r, pl.ds(u * 16, 16)] = z16
            return 0
        lax.fori_loop(0, NF, _zero, 0)

        def _edges(g, _):
            off = pl.multiple_of(g * 16, 16)
            sv = se_v[pl.ds(off, 16)]
            dv = de_v[pl.ds(off, 16)]
            asg = plsc.load_gather(as_v, [sv * H + head])
            adg = plsc.load_gather(ad_v, [dv * H + head])
            al = asg + adg
            al = jnp.where(al > 0, al, al * 0.2)
            ex = jnp.exp(al)
            plsc.addupdate_scatter(s_v, [dv, sv], ex)
            return 0
        lax.fori_loop(0, EPW // 16, _edges, 0)

        pltpu.sync_copy(s_v, s_o.at[wid])


def _gat_sc(a_src, a_dst, se, de):
    mesh = plsc.VectorSubcoreMesh(core_axis_name="c", subcore_axis_name="s")
    fn = pl.kernel(
        _gat_sc_body, mesh=mesh,
        out_type=jax.ShapeDtypeStruct((NWK, NF, NF), f32),
        scratch_types=[
            pltpu.VMEM((NF * H,), f32),
            pltpu.VMEM((NF * H,), f32),
            pltpu.VMEM((EPW,), jnp.int32),
            pltpu.VMEM((EPW,), jnp.int32),
            pltpu.VMEM((NF, NF), f32),
        ],
        compiler_params=pltpu.CompilerParams(needs_layout_passes=False),
    )
    return fn(a_src.reshape(NF * H), a_dst.reshape(NF * H), se, de)


def _lrelu(z, s):
    return jnp.maximum(z, z * s)


def _fold_prep(fe_r, wg_r, ne_r, w1a_r, sp_r, bias_r, w1b_r, g1_r, be1_r,
               at4_s, bt4_s):
    s0 = sp_r[0]
    s1m = sp_r[1]
    for w in range(2, NWK, 2):
        s0 = s0 + sp_r[w]
        s1m = s1m + sp_r[w + 1]
    den0 = jnp.sum(s0, axis=1, keepdims=True) + 1e-16
    den1 = jnp.sum(s1m, axis=1, keepdims=True) + 1e-16
    x = jnp.dot(fe_r[...], wg_r[...], preferred_element_type=f32)
    agg0 = jnp.dot(s0, x[:, :32], preferred_element_type=f32) / den0
    agg1 = jnp.dot(s1m, x[:, 32:], preferred_element_type=f32) / den1
    msg = jnp.concatenate([agg0, agg1], axis=1) + bias_r[...]
    b = jnp.dot(msg, w1b_r[...], preferred_element_type=f32)
    a = jnp.dot(ne_r[...], w1a_r[...], preferred_element_type=f32)
    am = jnp.mean(a, axis=0, keepdims=True)
    av = jnp.mean(a * a, axis=0, keepdims=True) - am * am
    bm = jnp.mean(b, axis=0, keepdims=True)
    bv = jnp.mean(b * b, axis=0, keepdims=True) - bm * bm
    sc1 = jax.lax.rsqrt(av + bv + 1e-5) * g1_r[...]
    at = (a - am) * sc1
    bt = (b - bm) * sc1 + be1_r[...]
    at4_s[...] = jnp.concatenate([at, at, at, at], axis=1).astype(jnp.bfloat16)
    bt4_s[...] = jnp.concatenate(
        [bt[0:64], bt[64:128], bt[128:192], bt[192:256]],
        axis=1).astype(jnp.bfloat16)


def _mlp_body(fe_r, wg_r, ne_r, w1a_r, sp_r, bias_r, w1b_r, g1_r, be1_r,
              w2u_r, g2_r, be2_r, w3b_r, b3_r, out_o,
              at4_s, bt4_s, stats_s, w2s_s, c2_s):
    i = pl.program_id(0)

    @pl.when(i == 0)
    def _():
        _fold_prep(fe_r, wg_r, ne_r, w1a_r, sp_r, bias_r, w1b_r, g1_r,
                   be1_r, at4_s, bt4_s)
        stats_s[...] = jnp.zeros((2, 128), f32)

    @pl.when(i < NBLK)
    def _():
        ablk = at4_s[pl.ds(i * SB, SB), :]
        z = ablk[:, None, :] + bt4_s[...][None, :, :]
        h1 = _lrelu(z, 0.01).reshape(SB * EMB, 4 * EMB)
        y = jnp.dot(h1, w2u_r[...], preferred_element_type=f32)
        sy = jnp.sum(y, axis=0, keepdims=True)
        syy = jnp.sum(y * y, axis=0, keepdims=True)
        stats_s[...] = stats_s[...] + jnp.concatenate([sy, syy], axis=0)

    @pl.when(i == NBLK)
    def _():
        st = stats_s[...]
        sy = (st[0:1, 0:32] + st[0:1, 32:64] + st[0:1, 64:96]
              + st[0:1, 96:128])
        syy = (st[1:2, 0:32] + st[1:2, 32:64] + st[1:2, 64:96]
               + st[1:2, 96:128])
        my = sy / NPAIR
        vy = syy / NPAIR - my * my
        s2 = g2_r[...] * jax.lax.rsqrt(vy + 1e-5)
        c2 = be2_r[...] - my * s2
        s2t = jnp.concatenate([s2, s2, s2, s2], axis=1)
        w2s_s[...] = (w2u_r[...].astype(f32) * s2t).astype(jnp.bfloat16)
        c2_s[...] = jnp.concatenate([c2, c2, c2, c2], axis=1)

    @pl.when(i >= NBLK)
    def _():
        j = i - NBLK
        ablk = at4_s[pl.ds(j * SB, SB), :]
        z = ablk[:, None, :] + bt4_s[...][None, :, :]
        h1 = _lrelu(z, 0.01).reshape(SB * EMB, 4 * EMB)
        yp = jnp.dot(h1, w2s_s[...], preferred_element_type=f32) + c2_s[...]
        h2 = _lrelu(yp, 0.01)
        v = jnp.dot(h2, w3b_r[...], preferred_element_type=f32) + b3_r[0, 0]
        v3 = v.reshape(SB, EMB, 4)
        out_o[...] = jnp.swapaxes(v3, 1, 2).reshape(SB, NF)


def _mlp(fe, wg, ne, w1a, sparts, bias_gat, w1b, g1, be1, w2u, g2, be2,
         w3b, b3):
    full = lambda s: pl.BlockSpec(s, lambda i: tuple(0 for _ in s))
    return pl.pallas_call(
        _mlp_body,
        grid=(2 * NBLK,),
        in_specs=[
            full((NF, EMB)),
            full((EMB, EMB)),
            full((BS, EMB)),
            full((EMB, EMB)),
            full((NWK, NF, NF)),
            full((1, EMB)),
            full((EMB, EMB)),
            full((1, EMB)),
            full((1, EMB)),
            full((4 * EMB, 128)),
            full((1, 32)),
            full((1, 32)),
            full((128, 4)),
            full((1, 1)),
        ],
        out_specs=pl.BlockSpec(
            (SB, NF), lambda i: (jnp.maximum(i - NBLK, 0), 0)),
        out_shape=jax.ShapeDtypeStruct((BS, NF), f32),
        scratch_shapes=[
            pltpu.VMEM((BS, 4 * EMB), jnp.bfloat16),
            pltpu.VMEM((EMB, 4 * EMB), jnp.bfloat16),
            pltpu.VMEM((2, 128), f32),
            pltpu.VMEM((4 * EMB, 128), jnp.bfloat16),
            pltpu.VMEM((1, 128), f32),
        ],
        compiler_params=pltpu.CompilerParams(
            dimension_semantics=("arbitrary",)),
    )(fe, wg, ne, w1a, sparts, bias_gat, w1b, g1, be1, w2u, g2, be2, w3b, b3)


def kernel(node_emb, feature_emb, relation_index, W_gat, att_src, att_dst,
           bias_gat, W1, b1, g1, be1, W2, b2, g2, be2, W3, b3):
    se = relation_index[0].astype(jnp.int32)
    de = relation_index[1].astype(jnp.int32)
    asw = jnp.zeros((EMB, H), f32).at[:32, 0].set(att_src[0]).at[32:, 1].set(att_src[1])
    adw = jnp.zeros((EMB, H), f32).at[:32, 0].set(att_dst[0]).at[32:, 1].set(att_dst[1])
    w1a = W1[:EMB]
    w1b = W1[EMB:]
    zpad = jnp.zeros((EMB, 32), f32)
    w2u = jnp.concatenate([
        jnp.concatenate([W2, zpad, zpad, zpad], axis=1),
        jnp.concatenate([zpad, W2, zpad, zpad], axis=1),
        jnp.concatenate([zpad, zpad, W2, zpad], axis=1),
        jnp.concatenate([zpad, zpad, zpad, W2], axis=1)],
        axis=0).astype(jnp.bfloat16)
    w3col = W3[:, 0]
    z32 = jnp.zeros((32,), f32)
    w3b = jnp.stack([
        jnp.concatenate([w3col, z32, z32, z32]),
        jnp.concatenate([z32, w3col, z32, z32]),
        jnp.concatenate([z32, z32, w3col, z32]),
        jnp.concatenate([z32, z32, z32, w3col])], axis=1)

    a_src, a_dst = _prep(feature_emb, W_gat @ asw, W_gat @ adw)
    sparts = _gat_sc(a_src, a_dst, se, de)

    out = _mlp(feature_emb, W_gat, node_emb, w1a, sparts,
               bias_gat.reshape(1, EMB), w1b,
               g1.reshape(1, EMB), be1.reshape(1, EMB), w2u,
               g2.reshape(1, 32), be2.reshape(1, 32), w3b, b3.reshape(1, 1))
    return out
